# baseline (speedup 1.0000x reference)
import functools

import jax
import jax.numpy as jnp
import numpy as np
from jax import lax
from jax.experimental import pallas as pl
from jax.experimental.pallas import tpu as pltpu
from jax.experimental.pallas import tpu_sc as plsc

_B, _A = 128, 32768
_W = 4096
_EPS = np.float32(1e-10)
_TINY = np.float32(1e-30)


def _threefry2x32(k1, k2, x0, x1):
    def rotl(x, d):
        return (x << np.uint32(d)) | (x >> np.uint32(32 - d))
    ks0, ks1 = np.uint32(k1), np.uint32(k2)
    ks2 = np.uint32(0x1BD11BDA) ^ ks0 ^ ks1
    rot = [np.array([13, 15, 26, 6], dtype=np.uint32),
           np.array([17, 29, 16, 24], dtype=np.uint32)]
    x0 = x0 + ks0
    x1 = x1 + ks1
    ks = [ks1, ks2, ks0]
    for i in range(5):
        for r in rot[i % 2]:
            x0 = x0 + x1
            x1 = rotl(x1, r)
            x1 = x0 ^ x1
        x0 = x0 + ks[0]
        x1 = x1 + ks[1] + np.uint32(i + 1)
        ks = [ks[1], ks[2], ks[0]]
    return x0, x1


def _np_uniform(key, n, minval, maxval):
    b1, b2 = _threefry2x32(key[0], key[1],
                           np.zeros(n, np.uint32), np.arange(n, dtype=np.uint32))
    bits = b1 ^ b2
    float_bits = (bits >> np.uint32(9)) | np.uint32(0x3F800000)
    f = float_bits.view(np.float32) - np.float32(1.0)
    minval, maxval = np.float32(minval), np.float32(maxval)
    return np.maximum(minval, f * (maxval - minval) + minval)


def _make_egs():
    seed_key = (np.uint32(0), np.uint32(42))
    b1, b2 = _threefry2x32(seed_key[0], seed_key[1],
                           np.zeros(2, np.uint32), np.arange(2, dtype=np.uint32))
    k_drop, k_samp = (b1[0], b2[0]), (b1[1], b2[1])
    n = _B * _A
    keep = _np_uniform(k_drop, n, 0.0, 1.0) > np.float32(0.5)
    tiny = np.float32(np.finfo(np.float32).tiny)
    u = _np_uniform(k_samp, n, tiny, 1.0).astype(np.float64)
    exp_gumbel = 1.0 / (-np.log(u))
    egs = np.where(keep, exp_gumbel, -exp_gumbel).astype(np.float32)
    return egs.reshape(_B, _A)


_EGS = _make_egs()
_EGA = np.ascontiguousarray(np.abs(_EGS).reshape(-1))


def _argmax_body(dist_ref, mask_ref, egs_ref, out_ref, mval_ref,
                 mv, iv, ml, il, has):
    j = pl.program_id(0)
    nj = pl.num_programs(0)

    @pl.when(j == 0)
    def _init():
        mv[...] = jnp.full((_B, 1), -jnp.inf, jnp.float32)
        iv[...] = jnp.zeros((_B, 1), jnp.int32)
        ml[...] = jnp.full((_B, 1), -jnp.inf, jnp.float32)
        il[...] = jnp.zeros((_B, 1), jnp.int32)
        has[...] = jnp.zeros((_B, 1), jnp.float32)

    d = dist_ref[...]
    m = mask_ref[...]
    egs = egs_ref[...]
    keep = egs > 0.0
    eg = jnp.abs(egs)

    lv = (d + _TINY) * eg
    sval = jnp.where(m != 0.0, _EPS, _TINY)
    v = jnp.where(keep, lv, sval * eg)

    gidx = lax.broadcasted_iota(jnp.int32, (_B, _W), 1) + j * _W
    big = jnp.int32(2147483647)

    bmv = jnp.max(v, axis=1, keepdims=True)
    biv = jnp.min(jnp.where(v == bmv, gidx, big), axis=1, keepdims=True)
    bml = jnp.max(lv, axis=1, keepdims=True)
    bil = jnp.min(jnp.where(lv == bml, gidx, big), axis=1, keepdims=True)
    bh = jnp.max(jnp.where(keep & (m != 0.0), 1.0, 0.0), axis=1, keepdims=True)

    upd_v = bmv > mv[...]
    mv[...] = jnp.where(upd_v, bmv, mv[...])
    iv[...] = jnp.where(upd_v, biv, iv[...])
    upd_l = bml > ml[...]
    ml[...] = jnp.where(upd_l, bml, ml[...])
    il[...] = jnp.where(upd_l, bil, il[...])
    has[...] = jnp.maximum(has[...], bh)

    @pl.when(j == nj - 1)
    def _fin():
        sel = has[...] > 0.0
        idx = jnp.where(sel, iv[...], il[...])
        rows = lax.broadcasted_iota(jnp.int32, (_B, 1), 0)
        out_ref[...] = rows * _A + idx
        mval_ref[...] = jnp.where(sel, mv[...], ml[...])


_argmax_call = pl.pallas_call(
    _argmax_body,
    grid=(_A // _W,),
    in_specs=[
        pl.BlockSpec((_B, _W), lambda j: (0, j)),
        pl.BlockSpec((_B, _W), lambda j: (0, j)),
        pl.BlockSpec((_B, _W), lambda j: (0, j)),
    ],
    out_specs=[
        pl.BlockSpec((_B, 1), lambda j: (0, 0)),
        pl.BlockSpec((_B, 1), lambda j: (0, 0)),
    ],
    out_shape=[
        jax.ShapeDtypeStruct((_B, 1), jnp.int32),
        jax.ShapeDtypeStruct((_B, 1), jnp.float32),
    ],
    scratch_shapes=[
        pltpu.VMEM((_B, 1), jnp.float32),
        pltpu.VMEM((_B, 1), jnp.int32),
        pltpu.VMEM((_B, 1), jnp.float32),
        pltpu.VMEM((_B, 1), jnp.int32),
        pltpu.VMEM((_B, 1), jnp.float32),
    ],
)

_NW_USED = 16
_BPW = _B // _NW_USED
_L = 16


@functools.lru_cache(maxsize=1)
def _get_sc_gather():
    @functools.partial(
        pl.kernel,
        mesh=plsc.VectorSubcoreMesh(core_axis_name="c", subcore_axis_name="s"),
        out_type=[
            jax.ShapeDtypeStruct((_B,), jnp.int32),
            jax.ShapeDtypeStruct((_B,), jnp.int32),
            jax.ShapeDtypeStruct((_B,), jnp.float32),
        ],
        scratch_types=[
            pltpu.VMEM((_BPW,), jnp.int32),
            pltpu.VMEM((_BPW,), jnp.int32),
            pltpu.VMEM((_BPW,), jnp.int32),
            pltpu.VMEM((_BPW,), jnp.float32),
            pltpu.SemaphoreType.DMA,
            pltpu.SemaphoreType.DMA,
            pltpu.SemaphoreType.DMA,
        ],
    )
    def _sc_gather(r_hbm, e_hbm, g_hbm, fidx_hbm, out_r, out_e, out_g,
                   idx_v, buf_r2, buf_e2, buf_g2, sem_r, sem_e, sem_g):
        num_cores = plsc.get_sparse_core_info().num_cores
        wid = lax.axis_index("s") * num_cores + lax.axis_index("c")

        @pl.when(wid < _NW_USED)
        def _():
            base = wid * _BPW
            pltpu.sync_copy(fidx_hbm.at[pl.ds(base, _BPW)], idx_v)
            cr = pltpu.async_copy(r_hbm.at[idx_v], buf_r2, sem_r)
            ce = pltpu.async_copy(e_hbm.at[idx_v], buf_e2, sem_e)
            cg = pltpu.async_copy(g_hbm.at[idx_v], buf_g2, sem_g)
            cr.wait()
            ce.wait()
            cg.wait()
            pltpu.sync_copy(buf_r2, out_r.at[pl.ds(base, _BPW)])
            pltpu.sync_copy(buf_e2, out_e.at[pl.ds(base, _BPW)])
            pltpu.sync_copy(buf_g2, out_g.at[pl.ds(base, _BPW)])

    return _sc_gather


def kernel(r_space, e_space, action_mask, action_dist):
    fidx, mval = _argmax_call(action_dist, action_mask, jnp.asarray(_EGS))
    next_r, next_e, eg_win = _get_sc_gather()(
        r_space.reshape(-1), e_space.reshape(-1), jnp.asarray(_EGA),
        fidx.reshape(_B))
    action_prob = mval.reshape(_B) / eg_win - _TINY
    return next_r, next_e, action_prob


# speedup vs baseline: 1.1830x; 1.1830x over previous
import functools

import jax
import jax.numpy as jnp
import numpy as np
from jax import lax
from jax.experimental import pallas as pl
from jax.experimental.pallas import tpu as pltpu
from jax.experimental.pallas import tpu_sc as plsc

_B, _A = 128, 32768
_W = 4096
_EPS = np.float32(1e-10)
_TINY = np.float32(1e-30)


def _threefry2x32(k1, k2, x0, x1):
    def rotl(x, d):
        return (x << np.uint32(d)) | (x >> np.uint32(32 - d))
    ks0, ks1 = np.uint32(k1), np.uint32(k2)
    ks2 = np.uint32(0x1BD11BDA) ^ ks0 ^ ks1
    rot = [np.array([13, 15, 26, 6], dtype=np.uint32),
           np.array([17, 29, 16, 24], dtype=np.uint32)]
    x0 = x0 + ks0
    x1 = x1 + ks1
    ks = [ks1, ks2, ks0]
    for i in range(5):
        for r in rot[i % 2]:
            x0 = x0 + x1
            x1 = rotl(x1, r)
            x1 = x0 ^ x1
        x0 = x0 + ks[0]
        x1 = x1 + ks[1] + np.uint32(i + 1)
        ks = [ks[1], ks[2], ks[0]]
    return x0, x1


def _np_uniform(key, n, minval, maxval):
    b1, b2 = _threefry2x32(key[0], key[1],
                           np.zeros(n, np.uint32), np.arange(n, dtype=np.uint32))
    bits = b1 ^ b2
    float_bits = (bits >> np.uint32(9)) | np.uint32(0x3F800000)
    f = float_bits.view(np.float32) - np.float32(1.0)
    minval, maxval = np.float32(minval), np.float32(maxval)
    return np.maximum(minval, f * (maxval - minval) + minval)


def _make_egs():
    seed_key = (np.uint32(0), np.uint32(42))
    b1, b2 = _threefry2x32(seed_key[0], seed_key[1],
                           np.zeros(2, np.uint32), np.arange(2, dtype=np.uint32))
    k_drop, k_samp = (b1[0], b2[0]), (b1[1], b2[1])
    n = _B * _A
    keep = _np_uniform(k_drop, n, 0.0, 1.0) > np.float32(0.5)
    tiny = np.float32(np.finfo(np.float32).tiny)
    u = _np_uniform(k_samp, n, tiny, 1.0).astype(np.float64)
    exp_gumbel = 1.0 / (-np.log(u))
    egs = np.where(keep, exp_gumbel, -exp_gumbel).astype(np.float32)
    return egs.reshape(_B, _A)


_EGS = _make_egs()

_KS1, _KS2 = (lambda b: (int(b[0][1]), int(b[1][1])))(
    _threefry2x32(np.uint32(0), np.uint32(42),
                  np.zeros(2, np.uint32), np.arange(2, dtype=np.uint32)))


def _eg_at(fidx):
    def rotl(x, d):
        return (x << np.uint32(d)) | (x >> np.uint32(32 - d))
    ks0 = np.uint32(_KS1)
    ks1 = np.uint32(_KS2)
    ks2 = np.uint32(0x1BD11BDA) ^ ks0 ^ ks1
    rot = [(13, 15, 26, 6), (17, 29, 16, 24)]
    x0 = jnp.zeros(fidx.shape, jnp.uint32) + ks0
    x1 = fidx.astype(jnp.uint32) + ks1
    ks = [ks1, ks2, ks0]
    for i in range(5):
        for r in rot[i % 2]:
            x0 = x0 + x1
            x1 = rotl(x1, r)
            x1 = x0 ^ x1
        x0 = x0 + ks[0]
        x1 = x1 + ks[1] + np.uint32(i + 1)
        ks = [ks[1], ks[2], ks[0]]
    bits = x0 ^ x1
    fb = (bits >> np.uint32(9)) | np.uint32(0x3F800000)
    f = lax.bitcast_convert_type(fb, jnp.float32) - np.float32(1.0)
    tiny = np.float32(np.finfo(np.float32).tiny)
    u = jnp.maximum(tiny, f * (np.float32(1.0) - tiny) + tiny)
    return np.float32(1.0) / (-jnp.log(u))


def _argmax_body(dist_ref, mask_ref, egs_ref, out_ref, mval_ref,
                 mv, iv, ml, il, has):
    j = pl.program_id(0)
    nj = pl.num_programs(0)

    @pl.when(j == 0)
    def _init():
        mv[...] = jnp.full((_B, 1), -jnp.inf, jnp.float32)
        iv[...] = jnp.zeros((_B, 1), jnp.int32)
        ml[...] = jnp.full((_B, 1), -jnp.inf, jnp.float32)
        il[...] = jnp.zeros((_B, 1), jnp.int32)
        has[...] = jnp.zeros((_B, 1), jnp.float32)

    d = dist_ref[...]
    m = mask_ref[...]
    egs = egs_ref[...]
    keep = egs > 0.0
    eg = jnp.abs(egs)

    lv = (d + _TINY) * eg
    sval = jnp.where(m != 0.0, _EPS, _TINY)
    v = jnp.where(keep, lv, sval * eg)

    gidx = lax.broadcasted_iota(jnp.int32, (_B, _W), 1) + j * _W
    big = jnp.int32(2147483647)

    bmv = jnp.max(v, axis=1, keepdims=True)
    biv = jnp.min(jnp.where(v == bmv, gidx, big), axis=1, keepdims=True)
    bml = jnp.max(lv, axis=1, keepdims=True)
    bil = jnp.min(jnp.where(lv == bml, gidx, big), axis=1, keepdims=True)
    bh = jnp.max(jnp.where(keep & (m != 0.0), 1.0, 0.0), axis=1, keepdims=True)

    upd_v = bmv > mv[...]
    mv[...] = jnp.where(upd_v, bmv, mv[...])
    iv[...] = jnp.where(upd_v, biv, iv[...])
    upd_l = bml > ml[...]
    ml[...] = jnp.where(upd_l, bml, ml[...])
    il[...] = jnp.where(upd_l, bil, il[...])
    has[...] = jnp.maximum(has[...], bh)

    @pl.when(j == nj - 1)
    def _fin():
        sel = has[...] > 0.0
        idx = jnp.where(sel, iv[...], il[...])
        rows = lax.broadcasted_iota(jnp.int32, (_B, 1), 0)
        out_ref[...] = rows * _A + idx
        mval_ref[...] = jnp.where(sel, mv[...], ml[...])


_argmax_call = pl.pallas_call(
    _argmax_body,
    grid=(_A // _W,),
    in_specs=[
        pl.BlockSpec((_B, _W), lambda j: (0, j)),
        pl.BlockSpec((_B, _W), lambda j: (0, j)),
        pl.BlockSpec((_B, _W), lambda j: (0, j)),
    ],
    out_specs=[
        pl.BlockSpec((_B, 1), lambda j: (0, 0)),
        pl.BlockSpec((_B, 1), lambda j: (0, 0)),
    ],
    out_shape=[
        jax.ShapeDtypeStruct((_B, 1), jnp.int32),
        jax.ShapeDtypeStruct((_B, 1), jnp.float32),
    ],
    scratch_shapes=[
        pltpu.VMEM((_B, 1), jnp.float32),
        pltpu.VMEM((_B, 1), jnp.int32),
        pltpu.VMEM((_B, 1), jnp.float32),
        pltpu.VMEM((_B, 1), jnp.int32),
        pltpu.VMEM((_B, 1), jnp.float32),
    ],
)

_NW_USED = 16
_BPW = _B // _NW_USED
_L = 16


@functools.lru_cache(maxsize=1)
def _get_sc_gather():
    @functools.partial(
        pl.kernel,
        mesh=plsc.VectorSubcoreMesh(core_axis_name="c", subcore_axis_name="s"),
        out_type=[
            jax.ShapeDtypeStruct((_B,), jnp.int32),
            jax.ShapeDtypeStruct((_B,), jnp.int32),
        ],
        scratch_types=[
            pltpu.VMEM((_BPW,), jnp.int32),
            pltpu.VMEM((_BPW,), jnp.int32),
            pltpu.VMEM((_BPW,), jnp.int32),
            pltpu.SemaphoreType.DMA,
            pltpu.SemaphoreType.DMA,
        ],
    )
    def _sc_gather(r_hbm, e_hbm, fidx_hbm, out_r, out_e,
                   idx_v, buf_r2, buf_e2, sem_r, sem_e):
        num_cores = plsc.get_sparse_core_info().num_cores
        wid = lax.axis_index("s") * num_cores + lax.axis_index("c")

        @pl.when(wid < _NW_USED)
        def _():
            base = wid * _BPW
            pltpu.sync_copy(fidx_hbm.at[pl.ds(base, _BPW)], idx_v)
            cr = pltpu.async_copy(r_hbm.at[idx_v], buf_r2, sem_r)
            ce = pltpu.async_copy(e_hbm.at[idx_v], buf_e2, sem_e)
            cr.wait()
            ce.wait()
            pltpu.sync_copy(buf_r2, out_r.at[pl.ds(base, _BPW)])
            pltpu.sync_copy(buf_e2, out_e.at[pl.ds(base, _BPW)])

    return _sc_gather


def kernel(r_space, e_space, action_mask, action_dist):
    fidx, mval = _argmax_call(action_dist, action_mask, jnp.asarray(_EGS))
    fidx = fidx.reshape(_B)
    next_r, next_e = _get_sc_gather()(
        r_space.reshape(-1), e_space.reshape(-1), fidx)
    action_prob = mval.reshape(_B) / _eg_at(fidx) - _TINY
    return next_r, next_e, action_prob
